# baseline (speedup 1.0000x reference)
import jax
import jax.numpy as jnp
from jax.experimental import pallas as pl
from jax.experimental.pallas import tpu as pltpu

_BM = 1024
_NBUF = 4


def _step(x_ref, o_ref):
    x = x_ref[...].astype(jnp.bfloat16)
    w = _step.w
    logits_t = jax.lax.dot_general(
        w, x,
        dimension_numbers=(((1,), (1,)), ((), ())),
        preferred_element_type=jnp.float32,
    )
    m = jnp.max(logits_t, axis=0, keepdims=True)
    e = jnp.exp(logits_t - m)
    o_ref[...] = e / jnp.sum(e, axis=0, keepdims=True)


def _outer(x_hbm, w_ref, o_hbm):
    w = w_ref[...].astype(jnp.bfloat16)
    M, K = x_hbm.shape
    E = w_ref.shape[0]

    def step(x_ref, o_ref):
        x = x_ref[...].astype(jnp.bfloat16)
        logits_t = jax.lax.dot_general(
            w, x,
            dimension_numbers=(((1,), (1,)), ((), ())),
            preferred_element_type=jnp.float32,
        )
        m = jnp.max(logits_t, axis=0, keepdims=True)
        e = jnp.exp(logits_t - m)
        o_ref[...] = e / jnp.sum(e, axis=0, keepdims=True)

    pipeline = pltpu.emit_pipeline(
        step,
        grid=(M // _BM,),
        in_specs=[pl.BlockSpec((_BM, K), lambda i: (i, 0),
                               pipeline_mode=pl.Buffered(buffer_count=_NBUF))],
        out_specs=[pl.BlockSpec((E, _BM), lambda i: (0, i))],
    )
    pipeline(x_hbm, o_hbm)


def kernel(inputs, W):
    M, K = inputs.shape
    E = W.shape[0]
    probs_t = pl.pallas_call(
        _outer,
        in_specs=[
            pl.BlockSpec(memory_space=pltpu.MemorySpace.HBM),
            pl.BlockSpec((E, K), lambda: (0, 0)),
        ],
        out_specs=pl.BlockSpec(memory_space=pltpu.MemorySpace.HBM),
        out_shape=jax.ShapeDtypeStruct((E, M), jnp.float32),
    )(inputs, W)
    return probs_t.T


# speedup vs baseline: 1.0431x; 1.0431x over previous
import jax
import jax.numpy as jnp
from jax.experimental import pallas as pl

_BM = 1024


def _router_block(x_ref, w_ref, o_ref):
    x = x_ref[...].astype(jnp.bfloat16)
    w = w_ref[...].astype(jnp.bfloat16)
    logits_t = jax.lax.dot_general(
        w, x,
        dimension_numbers=(((1,), (1,)), ((), ())),
        preferred_element_type=jnp.float32,
    )
    m = jnp.max(logits_t, axis=0, keepdims=True)
    e = jnp.exp(logits_t - m)
    o_ref[...] = e / jnp.sum(e, axis=0, keepdims=True)


def kernel(inputs, W):
    M, K = inputs.shape
    E = W.shape[0]
    grid = (M // _BM,)
    probs_t = pl.pallas_call(
        _router_block,
        grid=grid,
        in_specs=[
            pl.BlockSpec((_BM, K), lambda i: (i, 0)),
            pl.BlockSpec((E, K), lambda i: (0, 0)),
        ],
        out_specs=pl.BlockSpec((E, _BM), lambda i: (0, i)),
        out_shape=jax.ShapeDtypeStruct((E, M), jnp.float32),
    )(inputs, W)
    return probs_t.T
